# baseline (speedup 1.0000x reference)
import jax
import jax.numpy as jnp
from jax import lax
from jax.experimental import pallas as pl
from jax.experimental.pallas import tpu as pltpu
from jax.experimental.pallas import tpu_sc as plsc

BATCH = 1024
DIM = 64
NC = 2
NS = 16
L = 16
NW = NC * NS
BPW = BATCH // NW


def _body(x_hbm, lettre_hbm, pos_hbm, out_hbm, idx_v, rows_v, pos_v, sem):
    wid = lax.axis_index("s") * NC + lax.axis_index("c")
    base = wid * BPW
    pltpu.sync_copy(x_hbm.at[pl.ds(base, BPW)], idx_v)
    gather = pltpu.async_copy(lettre_hbm.at[idx_v], rows_v, sem)
    pltpu.sync_copy(pos_hbm.at[pl.ds(base, BPW)], pos_v)
    gather.wait()
    for i in range(BPW):
        for j in range(DIM // L):
            sl = pl.ds(j * L, L)
            rows_v[i, sl] = rows_v[i, sl] + pos_v[i, sl]
    pltpu.sync_copy(rows_v, out_hbm.at[pl.ds(base, BPW)])


@jax.jit
def kernel(x, embedLettre, embedPosition):
    mesh = plsc.VectorSubcoreMesh(core_axis_name="c", subcore_axis_name="s")
    k = pl.kernel(
        _body,
        mesh=mesh,
        out_type=jax.ShapeDtypeStruct((BATCH, DIM), jnp.float32),
        scratch_types=[
            pltpu.VMEM((BPW,), jnp.int32),
            pltpu.VMEM((BPW, DIM), jnp.float32),
            pltpu.VMEM((BPW, DIM), jnp.float32),
            pltpu.SemaphoreType.DMA,
        ],
        compiler_params=pltpu.CompilerParams(use_tc_tiling_on_sc=False),
    )
    return k(x, embedLettre, embedPosition)


# speedup vs baseline: 1.0032x; 1.0032x over previous
import jax
import jax.numpy as jnp
from jax import lax
from jax.experimental import pallas as pl
from jax.experimental.pallas import tpu as pltpu
from jax.experimental.pallas import tpu_sc as plsc

BATCH = 1024
DIM = 64
NC = 2
NS = 16
L = 16
NW = NC * NS
BPW = BATCH // NW


def _body(x_hbm, lettre_hbm, pos_hbm, out_hbm, idx_v, rows_v, pos_v, sem):
    wid = lax.axis_index("s") * NC + lax.axis_index("c")
    base = wid * BPW
    pltpu.sync_copy(x_hbm.at[pl.ds(base, BPW)], idx_v)
    pltpu.sync_copy(pos_hbm.at[pl.ds(base, BPW)], rows_v)
    pltpu.async_copy(lettre_hbm.at[idx_v], rows_v, sem, add=True).wait()
    pltpu.sync_copy(rows_v, out_hbm.at[pl.ds(base, BPW)])


@jax.jit
def kernel(x, embedLettre, embedPosition):
    mesh = plsc.VectorSubcoreMesh(core_axis_name="c", subcore_axis_name="s")
    k = pl.kernel(
        _body,
        mesh=mesh,
        out_type=jax.ShapeDtypeStruct((BATCH, DIM), jnp.float32),
        scratch_types=[
            pltpu.VMEM((BPW,), jnp.int32),
            pltpu.VMEM((BPW, DIM), jnp.float32),
            pltpu.VMEM((BPW, DIM), jnp.float32),
            pltpu.SemaphoreType.DMA,
        ],
        compiler_params=pltpu.CompilerParams(use_tc_tiling_on_sc=False),
    )
    return k(x, embedLettre, embedPosition)


# speedup vs baseline: 1.0070x; 1.0038x over previous
import jax
import jax.numpy as jnp
from jax import lax
from jax.experimental import pallas as pl
from jax.experimental.pallas import tpu as pltpu
from jax.experimental.pallas import tpu_sc as plsc

BATCH = 1024
DIM = 64
NC = 2
NS = 16
L = 16
NW = NC * NS
BPW = BATCH // NW


def _body(x_hbm, lettre_hbm, pos_hbm, out_hbm, idx_v, rows_v, pos_v, sem):
    wid = lax.axis_index("s") * NC + lax.axis_index("c")
    base = wid * BPW
    pltpu.sync_copy(x_hbm.at[pl.ds(base, BPW)], idx_v)
    pltpu.sync_copy(pos_hbm.at[pl.ds(base, BPW)], rows_v)
    pltpu.async_copy(lettre_hbm.at[idx_v], rows_v, sem, add=True).wait()
    pltpu.sync_copy(rows_v, out_hbm.at[pl.ds(base, BPW)])


@jax.jit
def kernel(x, embedLettre, embedPosition):
    mesh = plsc.VectorSubcoreMesh(core_axis_name="c", subcore_axis_name="s")
    k = pl.kernel(
        _body,
        mesh=mesh,
        out_type=jax.ShapeDtypeStruct((BATCH, DIM), jnp.float32),
        scratch_types=[
            pltpu.VMEM((BPW,), jnp.int32),
            pltpu.VMEM((BPW, DIM), jnp.float32),
            pltpu.VMEM((BPW, DIM), jnp.float32),
            pltpu.SemaphoreType.DMA,
        ],
        compiler_params=pltpu.CompilerParams(
            use_tc_tiling_on_sc=False,
            disable_bounds_checks=True,
            disable_semaphore_checks=True,
            skip_device_barrier=True,
        ),
    )
    return k(x, embedLettre, embedPosition)


# speedup vs baseline: 1.0211x; 1.0140x over previous
import jax
import jax.numpy as jnp
from jax import lax
from jax.experimental import pallas as pl
from jax.experimental.pallas import tpu as pltpu
from jax.experimental.pallas import tpu_sc as plsc

BATCH = 1024
DIM = 64
NC = 2
NS = 16
L = 16
NW = NC * NS
BPW = BATCH // NW


def _body(x_hbm, lettre_hbm, pos_hbm, out_hbm, idx_v, rows_v, sem, sem2):
    wid = lax.axis_index("s") * NC + lax.axis_index("c")
    base = wid * BPW
    idx_cp = pltpu.async_copy(x_hbm.at[pl.ds(base, BPW)], idx_v, sem2)
    pltpu.sync_copy(pos_hbm.at[pl.ds(base, BPW)], rows_v)
    idx_cp.wait()
    pltpu.async_copy(lettre_hbm.at[idx_v], rows_v, sem, add=True).wait()
    pltpu.sync_copy(rows_v, out_hbm.at[pl.ds(base, BPW)])


@jax.jit
def kernel(x, embedLettre, embedPosition):
    mesh = plsc.VectorSubcoreMesh(core_axis_name="c", subcore_axis_name="s")
    k = pl.kernel(
        _body,
        mesh=mesh,
        out_type=jax.ShapeDtypeStruct((BATCH, DIM), jnp.float32),
        scratch_types=[
            pltpu.VMEM((BPW,), jnp.int32),
            pltpu.VMEM((BPW, DIM), jnp.float32),
            pltpu.SemaphoreType.DMA,
            pltpu.SemaphoreType.DMA,
        ],
        compiler_params=pltpu.CompilerParams(
            use_tc_tiling_on_sc=False,
            disable_bounds_checks=True,
            disable_semaphore_checks=True,
            skip_device_barrier=True,
        ),
    )
    return k(x, embedLettre, embedPosition)
